# baseline (speedup 1.0000x reference)
import functools

import jax
import jax.numpy as jnp
from jax import lax
from jax.experimental import pallas as pl
from jax.experimental.pallas import tpu as pltpu
from jax.experimental.pallas import tpu_sc as plsc

_NDIG = 10
_LANES = 16
_GCHUNK = 128


@functools.cache
def _build(R, V):
    info = plsc.get_sparse_core_info()
    nc, ns = info.num_cores, info.num_subcores
    nw = nc * ns
    rw = R // nw
    assert R % nw == 0 and rw % _GCHUNK == 0
    chunks = rw // _GCHUNK
    jper = _GCHUNK // _LANES

    mesh = plsc.VectorSubcoreMesh(core_axis_name="c", subcore_axis_name="s")

    @functools.partial(
        pl.kernel,
        mesh=mesh,
        compiler_params=pltpu.CompilerParams(needs_layout_passes=False),
        out_type=jax.ShapeDtypeStruct((nw, rw), jnp.float32),
        scratch_types=[
            pltpu.VMEM((rw * _NDIG,), jnp.int32),
            pltpu.VMEM((rw,), jnp.int32),
            pltpu.VMEM((rw,), jnp.float32),
            pltpu.SemaphoreType.DMA,
        ],
    )
    def sc_kernel(x_hbm, vec_hbm, out_hbm, xv, idxv, outv, sem):
        wid = lax.axis_index("s") * nc + lax.axis_index("c")
        pltpu.sync_copy(x_hbm.at[pl.ds(wid * rw * _NDIG, rw * _NDIG)], xv)
        lanes = lax.iota(jnp.int32, _LANES)

        def chunk(g, carry):
            for j in range(jper):
                r10 = (g * _GCHUNK + j * _LANES + lanes) * _NDIG
                acc = plsc.load_gather(xv, [r10])
                for k in range(1, _NDIG):
                    acc = acc + (plsc.load_gather(xv, [r10 + k]) << (2 * k))
                idxv[pl.ds(g * _GCHUNK + j * _LANES, _LANES)] = acc
            pltpu.make_async_copy(
                vec_hbm.at[idxv.at[pl.ds(g * _GCHUNK, _GCHUNK)]],
                outv.at[pl.ds(g * _GCHUNK, _GCHUNK)],
                sem,
            ).start()
            return carry

        lax.fori_loop(0, chunks, chunk, 0)
        pltpu.make_async_copy(vec_hbm.at[pl.ds(0, rw)], outv, sem).wait()
        pltpu.sync_copy(outv, out_hbm.at[wid])

    return sc_kernel


def kernel(x, vec):
    n, b, n2 = x.shape
    R = n * b
    out = _build(R, vec.shape[0])(x.reshape(R * n2), vec)
    return out.reshape(n, b)


# speedup vs baseline: 2.1878x; 2.1878x over previous
import functools

import jax
import jax.numpy as jnp
from jax import lax
from jax.experimental import pallas as pl
from jax.experimental.pallas import tpu as pltpu
from jax.experimental.pallas import tpu_sc as plsc

_NDIG = 10
_LANES = 16
_GCHUNK = 128


@functools.cache
def _build(n, b, V):
    info = plsc.get_sparse_core_info()
    nc, ns = info.num_cores, info.num_subcores
    nw = nc * ns
    blk = b // nw
    assert b % nw == 0 and blk % _GCHUNK == 0
    gch = blk // _GCHUNK
    jper = _GCHUNK // _LANES

    mesh = plsc.VectorSubcoreMesh(core_axis_name="c", subcore_axis_name="s")

    @functools.partial(
        pl.kernel,
        mesh=mesh,
        compiler_params=pltpu.CompilerParams(
            needs_layout_passes=False, use_tc_tiling_on_sc=True
        ),
        out_type=jax.ShapeDtypeStruct((n * b,), jnp.float32),
        scratch_types=[
            pltpu.VMEM((n, _NDIG, blk), jnp.int32),
            pltpu.VMEM((blk,), jnp.int32),
            pltpu.VMEM((blk,), jnp.float32),
            pltpu.SemaphoreType.DMA,
        ],
    )
    def sc_kernel(xt_hbm, vec_hbm, out_hbm, xv, idxv, outv, sem):
        wid = lax.axis_index("s") * nc + lax.axis_index("c")
        b0 = wid * blk
        lanes = lax.iota(jnp.int32, _LANES)
        for i in range(n):
            pltpu.sync_copy(xt_hbm.at[i, :, pl.ds(b0, blk)], xv.at[i])
        for i in range(n):
            for g in range(gch):
                for j in range(jper):
                    c = g * _GCHUNK + j * _LANES
                    acc = xv[i, 0, pl.ds(c, _LANES)]
                    for k in range(1, _NDIG):
                        acc = acc + (xv[i, k, pl.ds(c, _LANES)] << (2 * k))
                    idxv[pl.ds(c, _LANES)] = acc
                pltpu.make_async_copy(
                    vec_hbm.at[idxv.at[pl.ds(g * _GCHUNK, _GCHUNK)]],
                    outv.at[pl.ds(g * _GCHUNK, _GCHUNK)],
                    sem,
                ).start()
            pltpu.make_async_copy(vec_hbm.at[pl.ds(0, blk)], outv, sem).wait()
            pltpu.sync_copy(outv, out_hbm.at[pl.ds(i * b + b0, blk)])

    return sc_kernel


def kernel(x, vec):
    n, b, n2 = x.shape
    xt = jnp.transpose(x, (0, 2, 1))
    out = _build(n, b, vec.shape[0])(xt, vec)
    return out.reshape(n, b)


# speedup vs baseline: 3.0709x; 1.4037x over previous
import functools

import jax
import jax.numpy as jnp
from jax import lax
from jax.experimental import pallas as pl
from jax.experimental.pallas import tpu as pltpu
from jax.experimental.pallas import tpu_sc as plsc

_NDIG = 10
_LANES = 16
_GCHUNK = 128


@functools.cache
def _build(n, b, V):
    info = plsc.get_sparse_core_info()
    nc, ns = info.num_cores, info.num_subcores
    nw = nc * ns
    blk = b // nw
    assert b % nw == 0 and blk % _GCHUNK == 0
    gch = blk // _GCHUNK
    jper = _GCHUNK // _LANES

    mesh = plsc.VectorSubcoreMesh(core_axis_name="c", subcore_axis_name="s")

    @functools.partial(
        pl.kernel,
        mesh=mesh,
        compiler_params=pltpu.CompilerParams(
            needs_layout_passes=False, use_tc_tiling_on_sc=True
        ),
        out_type=jax.ShapeDtypeStruct((n, b), jnp.float32),
        scratch_types=[
            pltpu.VMEM((n, _NDIG, blk), jnp.int32),
            pltpu.VMEM((n * blk,), jnp.int32),
            pltpu.VMEM((n * blk,), jnp.float32),
            pltpu.SemaphoreType.DMA((n,)),
            pltpu.SemaphoreType.DMA,
            pltpu.SemaphoreType.DMA,
        ],
    )
    def sc_kernel(xt_hbm, vec_hbm, out_hbm, xv, idxv, outv, semx, semg, semo):
        wid = lax.axis_index("s") * nc + lax.axis_index("c")
        b0 = wid * blk
        for i in range(n):
            pltpu.make_async_copy(
                xt_hbm.at[i, :, pl.ds(b0, blk)], xv.at[i], semx.at[i]
            ).start()
        for i in range(n):
            pltpu.make_async_copy(
                xt_hbm.at[i, :, pl.ds(b0, blk)], xv.at[i], semx.at[i]
            ).wait()
            for g in range(gch):
                for j in range(jper):
                    c = g * _GCHUNK + j * _LANES
                    acc = xv[i, 0, pl.ds(c, _LANES)]
                    for k in range(1, _NDIG):
                        acc = acc + (xv[i, k, pl.ds(c, _LANES)] << (2 * k))
                    idxv[pl.ds(i * blk + c, _LANES)] = acc
                pltpu.make_async_copy(
                    vec_hbm.at[idxv.at[pl.ds(i * blk + g * _GCHUNK, _GCHUNK)]],
                    outv.at[pl.ds(i * blk + g * _GCHUNK, _GCHUNK)],
                    semg,
                ).start()
        pltpu.make_async_copy(vec_hbm.at[pl.ds(0, n * blk)], outv, semg).wait()
        for i in range(n):
            pltpu.make_async_copy(
                outv.at[pl.ds(i * blk, blk)], out_hbm.at[i, pl.ds(b0, blk)], semo
            ).start()
        for i in range(n):
            pltpu.make_async_copy(
                outv.at[pl.ds(i * blk, blk)], out_hbm.at[i, pl.ds(b0, blk)], semo
            ).wait()

    return sc_kernel


def kernel(x, vec):
    n, b, n2 = x.shape
    xt = jnp.transpose(x, (0, 2, 1))
    return _build(n, b, vec.shape[0])(xt, vec)
